# baseline (speedup 1.0000x reference)
import functools

import jax
import jax.numpy as jnp
from jax import lax
from jax.experimental import pallas as pl
from jax.experimental.pallas import tpu as pltpu
from jax.experimental.pallas import tpu_sc as plsc

N = 160000
D = 256
DIM = 10000
DIMP = 10240
NC = 2
NS = 16
L = 16
EPT = N // NS
CHUNKS = EPT // L
SLICE = DIMP // NS
OUT_PER = DIMP // (NC * NS)
GCH = 40
NGCH = OUT_PER // GCH
U = 4

_NEG_INF = float(jnp.finfo(jnp.float32).min)


def _any_lane(mask):
    cnt = plsc.all_reduce_population_count(mask)
    return jnp.squeeze(lax.slice(cnt, (0,), (1,))) > 0


def _lex_update(tseg_v, pmax_v, tmp_v, idx, tv, pos, act):
    plsc.store_scatter(tmp_v, [idx], pos, mask=act)
    w = plsc.load_gather(tmp_v, [idx])
    win = jnp.logical_and(act, w == pos)
    ct = plsc.load_gather(tseg_v, [idx])
    cp = plsc.load_gather(pmax_v, [idx])
    bet = jnp.logical_or(tv > ct,
                         jnp.logical_and(tv == ct, pos > cp))
    wr = jnp.logical_and(win, bet)
    plsc.store_scatter(tseg_v, [idx], tv, mask=wr)
    plsc.store_scatter(pmax_v, [idx], pos, mask=wr)
    return jnp.logical_and(jnp.logical_and(act, jnp.logical_not(win)), bet)


def _body(msg_hbm, idx_hbm, t_hbm, ninf_hbm, neg1_hbm, out_hbm,
          idx_v, t_v, tseg_v, pmax_v, tmp_v, red_v,
          red_p, amax_v, safe_v, rows0_v, rows1_v, zrow_v,
          sem_in, sem0, sem1, sp_t, sp_p):
    c = lax.axis_index("c")
    s = lax.axis_index("s")
    base_in = s * EPT
    lane = lax.iota(jnp.int32, L)

    HG = (CHUNKS // U) // 2
    HE = HG * U * L
    pltpu.async_copy(idx_hbm.at[pl.ds(base_in, HE)],
                     idx_v.at[pl.ds(0, HE)], sem_in)
    pltpu.async_copy(t_hbm.at[pl.ds(base_in, HE)],
                     t_v.at[pl.ds(0, HE)], sem_in)
    pltpu.async_copy(ninf_hbm, tseg_v, sem_in)
    pltpu.async_copy(neg1_hbm, tmp_v, sem_in)
    pltpu.async_copy(idx_hbm.at[pl.ds(base_in + HE, EPT - HE)],
                     idx_v.at[pl.ds(HE, EPT - HE)], sem1)
    pltpu.async_copy(t_hbm.at[pl.ds(base_in + HE, EPT - HE)],
                     t_v.at[pl.ds(HE, EPT - HE)], sem1)
    pltpu.make_async_copy(idx_hbm.at[pl.ds(base_in, HE)],
                          idx_v.at[pl.ds(0, HE)], sem_in).wait()
    pltpu.make_async_copy(t_hbm.at[pl.ds(base_in, HE)],
                          t_v.at[pl.ds(0, HE)], sem_in).wait()
    pltpu.make_async_copy(ninf_hbm, tseg_v, sem_in).wait()
    pltpu.make_async_copy(neg1_hbm, tmp_v, sem_in).wait()

    all_act = lane >= 0

    def _slow_fix(idx, tv, pos, rem):
        def cond(m):
            return _any_lane(m > 0)

        def body(m):
            return _lex_update(
                tseg_v, pmax_v, tmp_v, idx, tv, pos, m > 0
            ).astype(jnp.int32)

        lax.while_loop(cond, body, rem.astype(jnp.int32))

    def group(i, _):
        base = i * (U * L)
        idxs, tvs, poss, ws = [], [], [], []
        for j in range(U):
            idx = idx_v[pl.ds(base + j * L, L)]
            tv = t_v[pl.ds(base + j * L, L)]
            pos = base_in + base + j * L + lane
            plsc.store_scatter(tmp_v, [idx], pos)
            idxs.append(idx)
            tvs.append(tv)
            poss.append(pos)
        for j in range(U):
            ws.append(plsc.load_gather(tmp_v, [idxs[j]]))
        rems = []
        for j in range(U):
            win = ws[j] == poss[j]
            ct = plsc.load_gather(tseg_v, [idxs[j]])
            cp = plsc.load_gather(pmax_v, [idxs[j]])
            bet = jnp.logical_or(
                tvs[j] > ct,
                jnp.logical_and(tvs[j] == ct, poss[j] > cp))
            wr = jnp.logical_and(win, bet)
            plsc.store_scatter(tseg_v, [idxs[j]], tvs[j], mask=wr)
            plsc.store_scatter(pmax_v, [idxs[j]], poss[j], mask=wr)
            rems.append(jnp.logical_and(jnp.logical_not(win), bet))
        any_rem = rems[0]
        for j in range(1, U):
            any_rem = jnp.logical_or(any_rem, rems[j])

        @pl.when(_any_lane(any_rem))
        def _slow():
            for j in range(U):
                _slow_fix(idxs[j], tvs[j], poss[j], rems[j])

        return _

    lax.fori_loop(0, HG, group, None)
    pltpu.make_async_copy(idx_hbm.at[pl.ds(base_in + HE, EPT - HE)],
                          idx_v.at[pl.ds(HE, EPT - HE)], sem1).wait()
    pltpu.make_async_copy(t_hbm.at[pl.ds(base_in + HE, EPT - HE)],
                          t_v.at[pl.ds(HE, EPT - HE)], sem1).wait()
    lax.fori_loop(HG, CHUNKS // U, group, None)

    for i in range(U * (CHUNKS // U), CHUNKS):
        idx = idx_v[pl.ds(i * L, L)]
        tv = t_v[pl.ds(i * L, L)]
        pos = base_in + i * L + lane
        rem = _lex_update(tseg_v, pmax_v, tmp_v, idx, tv, pos, all_act)

        @pl.when(_any_lane(rem))
        def _slow_tail():
            _slow_fix(idx, tv, pos, rem)

    pltpu.sync_copy(tseg_v, sp_t.at[s])
    pltpu.sync_copy(pmax_v, sp_p.at[s])
    plsc.subcore_barrier()
    woff = s * SLICE + c * 256
    pltpu.sync_copy(sp_t.at[:, pl.ds(woff, 384)], red_v)
    pltpu.sync_copy(sp_p.at[:, pl.ds(woff, 384)], red_p)
    coff = c * 64

    def red(j, _):
        ta = red_v[0, pl.ds(coff + j * L, L)]
        pa = red_p[0, pl.ds(coff + j * L, L)]
        for k in range(1, NS):
            tk = red_v[k, pl.ds(coff + j * L, L)]
            pk = red_p[k, pl.ds(coff + j * L, L)]
            b = jnp.logical_or(tk > ta,
                               jnp.logical_and(tk == ta, pk > pa))
            ta = jnp.where(b, tk, ta)
            pa = jnp.where(b, pk, pa)
        amax_v[pl.ds(j * L, L)] = jnp.where(
            ta == _NEG_INF, jnp.int32(-1), pa)
        return _

    lax.fori_loop(0, OUT_PER // L, red, None)

    obase = s * SLICE + c * OUT_PER
    for j in range(OUT_PER // L):
        a = amax_v[pl.ds(j * L, L)]
        safe_v[pl.ds(j * L, L)] = jnp.maximum(a, 0)
    for k in range(D // L):
        zrow_v[0, pl.ds(k * L, L)] = jnp.zeros((L,), jnp.float32)

    rows = [rows0_v, rows1_v]
    sems = [sem0, sem1]

    def in_range(cc):
        return obase + cc * GCH + GCH <= DIM

    def issue(cc):
        pltpu.async_copy(
            msg_hbm.at[safe_v.at[pl.ds(cc * GCH, GCH)]],
            rows[cc % 2], sems[cc % 2])

    def drain(cc):
        pltpu.make_async_copy(
            msg_hbm.at[safe_v.at[pl.ds(cc * GCH, GCH)]],
            rows[cc % 2], sems[cc % 2]).wait()

    for cc in range(2):
        @pl.when(in_range(cc))
        def _pre(cc=cc):
            issue(cc)

    for cc in range(NGCH):
        @pl.when(in_range(cc))
        def _chunk(cc=cc):
            drain(cc)
            if cc + 2 < NGCH:
                @pl.when(in_range(cc + 2))
                def _nxt():
                    issue(cc + 2)
            pltpu.sync_copy(rows[cc % 2],
                            out_hbm.at[pl.ds(obase + cc * GCH, GCH)])

    def zfix(j, _):
        a16 = amax_v[pl.ds(j * L, L)]

        @pl.when(_any_lane(a16 < 0))
        def _zero():
            def zrow(r, _):
                a_r = jnp.sum(jnp.where(lane == r, a16, 0))
                row = obase + j * L + r

                @pl.when(jnp.logical_and(a_r < 0, row < DIM))
                def _wr():
                    pltpu.sync_copy(zrow_v, out_hbm.at[pl.ds(row, 1)])

                return _

            lax.fori_loop(0, L, zrow, None)

        return _

    lax.fori_loop(0, OUT_PER // L, zfix, None)


@functools.partial(jax.jit, static_argnums=())
def kernel(msg, index, t, dim_size):
    del dim_size
    ninf = jnp.full((DIMP,), _NEG_INF, dtype=jnp.float32)
    neg1 = jnp.full((DIMP,), -1, dtype=jnp.int32)

    mesh = plsc.VectorSubcoreMesh(
        core_axis_name="c", subcore_axis_name="s",
        num_cores=NC, num_subcores=NS)
    run = pl.kernel(
        _body,
        out_type=jax.ShapeDtypeStruct((DIM, D), jnp.float32),
        mesh=mesh,
        compiler_params=pltpu.CompilerParams(needs_layout_passes=False),
        scratch_types=[
            pltpu.VMEM((EPT,), jnp.int32),
            pltpu.VMEM((EPT,), jnp.float32),
            pltpu.VMEM((DIMP,), jnp.float32),
            pltpu.VMEM((DIMP,), jnp.int32),
            pltpu.VMEM((DIMP,), jnp.int32),
            pltpu.VMEM((NS, 384), jnp.float32),
            pltpu.VMEM((NS, 384), jnp.int32),
            pltpu.VMEM((OUT_PER,), jnp.int32),
            pltpu.VMEM((OUT_PER,), jnp.int32),
            pltpu.VMEM((GCH, D), jnp.float32),
            pltpu.VMEM((GCH, D), jnp.float32),
            pltpu.VMEM((1, D), jnp.float32),
            pltpu.SemaphoreType.DMA,
            pltpu.SemaphoreType.DMA,
            pltpu.SemaphoreType.DMA,
            pltpu.VMEM_SHARED((NS, DIMP), jnp.float32),
            pltpu.VMEM_SHARED((NS, DIMP), jnp.int32),
        ],
    )
    return run(msg, index, t, ninf, neg1)


# speedup vs baseline: 1.0172x; 1.0172x over previous
import functools

import jax
import jax.numpy as jnp
from jax import lax
from jax.experimental import pallas as pl
from jax.experimental.pallas import tpu as pltpu
from jax.experimental.pallas import tpu_sc as plsc

N = 160000
D = 256
DIM = 10000
DIMP = 10240
NC = 2
NS = 16
L = 16
EPT = N // NS
CHUNKS = EPT // L
SLICE = DIMP // NS
OUT_PER = DIMP // (NC * NS)
GCH = 80
NGCH = OUT_PER // GCH
U = 4

_NEG_INF = float(jnp.finfo(jnp.float32).min)


def _any_lane(mask):
    cnt = plsc.all_reduce_population_count(mask)
    return jnp.squeeze(lax.slice(cnt, (0,), (1,))) > 0


def _lex_update(tseg_v, pmax_v, tmp_v, idx, tv, pos, act):
    plsc.store_scatter(tmp_v, [idx], pos, mask=act)
    w = plsc.load_gather(tmp_v, [idx])
    win = jnp.logical_and(act, w == pos)
    ct = plsc.load_gather(tseg_v, [idx])
    cp = plsc.load_gather(pmax_v, [idx])
    bet = jnp.logical_or(tv > ct,
                         jnp.logical_and(tv == ct, pos > cp))
    wr = jnp.logical_and(win, bet)
    plsc.store_scatter(tseg_v, [idx], tv, mask=wr)
    plsc.store_scatter(pmax_v, [idx], pos, mask=wr)
    return jnp.logical_and(jnp.logical_and(act, jnp.logical_not(win)), bet)


def _body(msg_hbm, idx_hbm, t_hbm, ninf_hbm, neg1_hbm, out_hbm,
          idx_v, t_v, tseg_v, pmax_v, tmp_v, red_v,
          red_p, amax_v, safe_v, rows0_v, rows1_v, zrow_v,
          sem_in, sem0, sem1, sp_t, sp_p):
    c = lax.axis_index("c")
    s = lax.axis_index("s")
    base_in = s * EPT
    lane = lax.iota(jnp.int32, L)

    pltpu.async_copy(idx_hbm.at[pl.ds(base_in, EPT)], idx_v, sem_in)
    pltpu.async_copy(t_hbm.at[pl.ds(base_in, EPT)], t_v, sem_in)
    pltpu.async_copy(ninf_hbm, tseg_v, sem_in)
    cp_last = pltpu.async_copy(neg1_hbm, tmp_v, sem_in)
    pltpu.make_async_copy(idx_hbm.at[pl.ds(base_in, EPT)], idx_v,
                          sem_in).wait()
    pltpu.make_async_copy(t_hbm.at[pl.ds(base_in, EPT)], t_v,
                          sem_in).wait()
    pltpu.make_async_copy(ninf_hbm, tseg_v, sem_in).wait()
    cp_last.wait()

    all_act = lane >= 0

    def _slow_fix(idx, tv, pos, rem):
        def cond(m):
            return _any_lane(m > 0)

        def body(m):
            return _lex_update(
                tseg_v, pmax_v, tmp_v, idx, tv, pos, m > 0
            ).astype(jnp.int32)

        lax.while_loop(cond, body, rem.astype(jnp.int32))

    def group(i, _):
        base = i * (U * L)
        idxs, tvs, poss, ws = [], [], [], []
        for j in range(U):
            idx = idx_v[pl.ds(base + j * L, L)]
            tv = t_v[pl.ds(base + j * L, L)]
            pos = base_in + base + j * L + lane
            plsc.store_scatter(tmp_v, [idx], pos)
            idxs.append(idx)
            tvs.append(tv)
            poss.append(pos)
        for j in range(U):
            ws.append(plsc.load_gather(tmp_v, [idxs[j]]))
        rems = []
        for j in range(U):
            win = ws[j] == poss[j]
            ct = plsc.load_gather(tseg_v, [idxs[j]])
            cp = plsc.load_gather(pmax_v, [idxs[j]])
            bet = jnp.logical_or(
                tvs[j] > ct,
                jnp.logical_and(tvs[j] == ct, poss[j] > cp))
            wr = jnp.logical_and(win, bet)
            plsc.store_scatter(tseg_v, [idxs[j]], tvs[j], mask=wr)
            plsc.store_scatter(pmax_v, [idxs[j]], poss[j], mask=wr)
            rems.append(jnp.logical_and(jnp.logical_not(win), bet))
        any_rem = rems[0]
        for j in range(1, U):
            any_rem = jnp.logical_or(any_rem, rems[j])

        @pl.when(_any_lane(any_rem))
        def _slow():
            for j in range(U):
                _slow_fix(idxs[j], tvs[j], poss[j], rems[j])

        return _

    lax.fori_loop(0, CHUNKS // U, group, None)

    for i in range(U * (CHUNKS // U), CHUNKS):
        idx = idx_v[pl.ds(i * L, L)]
        tv = t_v[pl.ds(i * L, L)]
        pos = base_in + i * L + lane
        rem = _lex_update(tseg_v, pmax_v, tmp_v, idx, tv, pos, all_act)

        @pl.when(_any_lane(rem))
        def _slow_tail():
            _slow_fix(idx, tv, pos, rem)

    pltpu.sync_copy(tseg_v, sp_t.at[s])
    pltpu.sync_copy(pmax_v, sp_p.at[s])
    plsc.subcore_barrier()
    woff = s * SLICE + c * 256
    pltpu.sync_copy(sp_t.at[:, pl.ds(woff, 384)], red_v)
    pltpu.sync_copy(sp_p.at[:, pl.ds(woff, 384)], red_p)
    coff = c * 64

    def red(j, _):
        ta = red_v[0, pl.ds(coff + j * L, L)]
        pa = red_p[0, pl.ds(coff + j * L, L)]
        for k in range(1, NS):
            tk = red_v[k, pl.ds(coff + j * L, L)]
            pk = red_p[k, pl.ds(coff + j * L, L)]
            b = jnp.logical_or(tk > ta,
                               jnp.logical_and(tk == ta, pk > pa))
            ta = jnp.where(b, tk, ta)
            pa = jnp.where(b, pk, pa)
        amax_v[pl.ds(j * L, L)] = jnp.where(
            ta == _NEG_INF, jnp.int32(-1), pa)
        return _

    lax.fori_loop(0, OUT_PER // L, red, None)

    obase = s * SLICE + c * OUT_PER
    for j in range(OUT_PER // L):
        a = amax_v[pl.ds(j * L, L)]
        safe_v[pl.ds(j * L, L)] = jnp.maximum(a, 0)
    for k in range(D // L):
        zrow_v[0, pl.ds(k * L, L)] = jnp.zeros((L,), jnp.float32)

    rows = [rows0_v, rows1_v]
    sems = [sem0, sem1]

    def in_range(cc):
        return obase + cc * GCH + GCH <= DIM

    def issue(cc):
        pltpu.async_copy(
            msg_hbm.at[safe_v.at[pl.ds(cc * GCH, GCH)]],
            rows[cc % 2], sems[cc % 2])

    def drain(cc):
        pltpu.make_async_copy(
            msg_hbm.at[safe_v.at[pl.ds(cc * GCH, GCH)]],
            rows[cc % 2], sems[cc % 2]).wait()

    for cc in range(2):
        @pl.when(in_range(cc))
        def _pre(cc=cc):
            issue(cc)

    for cc in range(NGCH):
        @pl.when(in_range(cc))
        def _chunk(cc=cc):
            drain(cc)
            if cc + 2 < NGCH:
                @pl.when(in_range(cc + 2))
                def _nxt():
                    issue(cc + 2)
            pltpu.sync_copy(rows[cc % 2],
                            out_hbm.at[pl.ds(obase + cc * GCH, GCH)])

    def zfix(j, _):
        a16 = amax_v[pl.ds(j * L, L)]

        @pl.when(_any_lane(a16 < 0))
        def _zero():
            def zrow(r, _):
                a_r = jnp.sum(jnp.where(lane == r, a16, 0))
                row = obase + j * L + r

                @pl.when(jnp.logical_and(a_r < 0, row < DIM))
                def _wr():
                    pltpu.sync_copy(zrow_v, out_hbm.at[pl.ds(row, 1)])

                return _

            lax.fori_loop(0, L, zrow, None)

        return _

    lax.fori_loop(0, OUT_PER // L, zfix, None)


@functools.partial(jax.jit, static_argnums=())
def kernel(msg, index, t, dim_size):
    del dim_size
    ninf = jnp.full((DIMP,), _NEG_INF, dtype=jnp.float32)
    neg1 = jnp.full((DIMP,), -1, dtype=jnp.int32)

    mesh = plsc.VectorSubcoreMesh(
        core_axis_name="c", subcore_axis_name="s",
        num_cores=NC, num_subcores=NS)
    run = pl.kernel(
        _body,
        out_type=jax.ShapeDtypeStruct((DIM, D), jnp.float32),
        mesh=mesh,
        compiler_params=pltpu.CompilerParams(needs_layout_passes=False),
        scratch_types=[
            pltpu.VMEM((EPT,), jnp.int32),
            pltpu.VMEM((EPT,), jnp.float32),
            pltpu.VMEM((DIMP,), jnp.float32),
            pltpu.VMEM((DIMP,), jnp.int32),
            pltpu.VMEM((DIMP,), jnp.int32),
            pltpu.VMEM((NS, 384), jnp.float32),
            pltpu.VMEM((NS, 384), jnp.int32),
            pltpu.VMEM((OUT_PER,), jnp.int32),
            pltpu.VMEM((OUT_PER,), jnp.int32),
            pltpu.VMEM((GCH, D), jnp.float32),
            pltpu.VMEM((GCH, D), jnp.float32),
            pltpu.VMEM((1, D), jnp.float32),
            pltpu.SemaphoreType.DMA,
            pltpu.SemaphoreType.DMA,
            pltpu.SemaphoreType.DMA,
            pltpu.VMEM_SHARED((NS, DIMP), jnp.float32),
            pltpu.VMEM_SHARED((NS, DIMP), jnp.int32),
        ],
    )
    return run(msg, index, t, ninf, neg1)
